# baseline (speedup 1.0000x reference)
import functools

import jax
import jax.numpy as jnp
from jax import lax
from jax.experimental import pallas as pl
from jax.experimental.pallas import tpu as pltpu
from jax.experimental.pallas import tpu_sc as plsc

N = 10000
E = 320000
D = 128
H = 16

NC = 2
NS = 16
CHUNK = 128
E_PAD = 327680
CH_TOT = E_PAD // CHUNK
CH_SC = CH_TOT // NC
CH_SUB = CH_SC // NS
GRP = 16
N_GRP = CH_SUB // GRP
NA = N + 8
R_MAIN = 632
R_LAST = N - (NS - 1) * R_MAIN
RZ_LAST = NA - (NS - 1) * R_MAIN

_mesh = plsc.VectorSubcoreMesh(core_axis_name="c", subcore_axis_name="s")
_sc_params = pltpu.CompilerParams(use_tc_tiling_on_sc=False)


def _zero_acc(s, zbuf, acc):
    def zero_row(i, carry):
        zbuf[i, :] = jnp.zeros((H,), jnp.float32)
        return carry

    lax.fori_loop(0, R_MAIN, zero_row, 0)

    @pl.when(s < NS - 1)
    def _():
        pltpu.sync_copy(zbuf, acc.at[pl.ds(s * R_MAIN, R_MAIN)])

    @pl.when(s == NS - 1)
    def _():
        pltpu.sync_copy(zbuf.at[pl.ds(0, RZ_LAST)],
                        acc.at[pl.ds((NS - 1) * R_MAIN, RZ_LAST)])


def _copy_out(c, s, acc, out_hbm):
    @pl.when(s < NS - 1)
    def _():
        pltpu.sync_copy(acc.at[pl.ds(s * R_MAIN, R_MAIN)],
                        out_hbm.at[c, pl.ds(s * R_MAIN, R_MAIN)])

    @pl.when(s == NS - 1)
    def _():
        pltpu.sync_copy(acc.at[pl.ds((NS - 1) * R_MAIN, R_LAST)],
                        out_hbm.at[c, pl.ds((NS - 1) * R_MAIN, R_LAST)])


@functools.partial(
    pl.kernel,
    out_type=jax.ShapeDtypeStruct((NC, N, H), jnp.float32),
    mesh=_mesh,
    compiler_params=_sc_params,
    scratch_types=[
        pltpu.VMEM((CH_SUB, CHUNK), jnp.int32),
        pltpu.VMEM((CH_SUB, CHUNK), jnp.int32),
        pltpu.VMEM((GRP, CHUNK, H), jnp.float32),
        pltpu.VMEM((R_MAIN, H), jnp.float32),
        pltpu.VMEM_SHARED((NA, H), jnp.float32),
        pltpu.VMEM_SHARED((N, H), jnp.float32),
        pltpu.SemaphoreType.DMA,
        pltpu.SemaphoreType.DMA,
    ],
)
def _segsum(src_hbm, dst_hbm, y_hbm, out_hbm, sidx, didx, rows, zbuf, acc,
            ytab, gsem, ssem):
    c = lax.axis_index("c")
    s = lax.axis_index("s")
    _zero_acc(s, zbuf, acc)

    @pl.when(s < NS - 1)
    def _():
        pltpu.sync_copy(y_hbm.at[pl.ds(s * R_MAIN, R_MAIN)],
                        ytab.at[pl.ds(s * R_MAIN, R_MAIN)])

    @pl.when(s == NS - 1)
    def _():
        pltpu.sync_copy(y_hbm.at[pl.ds((NS - 1) * R_MAIN, R_LAST)],
                        ytab.at[pl.ds((NS - 1) * R_MAIN, R_LAST)])

    chunk_off = c * CH_SC + s * CH_SUB
    pltpu.sync_copy(src_hbm.at[pl.ds(chunk_off, CH_SUB)], sidx)
    pltpu.sync_copy(dst_hbm.at[pl.ds(chunk_off, CH_SUB)], didx)
    plsc.subcore_barrier()

    def group(g, carry):
        kb = g * GRP
        gd = [pltpu.async_copy(ytab.at[sidx.at[kb + b]], rows.at[b], gsem)
              for b in range(GRP)]
        sd = []
        for b in range(GRP):
            gd[b].wait()
            sd.append(pltpu.async_copy(rows.at[b], acc.at[didx.at[kb + b]],
                                       ssem, add=True))
        for d in sd:
            d.wait()
        return carry

    lax.fori_loop(0, N_GRP, group, 0)
    plsc.subcore_barrier()
    _copy_out(c, s, acc, out_hbm)


@functools.partial(
    pl.kernel,
    out_type=jax.ShapeDtypeStruct((NC, N, H), jnp.float32),
    mesh=_mesh,
    compiler_params=_sc_params,
    scratch_types=[
        pltpu.VMEM((CH_SUB, CHUNK), jnp.int32),
        pltpu.VMEM((CHUNK, H), jnp.float32),
        pltpu.VMEM((R_MAIN, H), jnp.float32),
        pltpu.VMEM_SHARED((NA, H), jnp.float32),
        pltpu.SemaphoreType.DMA,
    ],
)
def _degsum(dst_hbm, out_hbm, didx, ones, zbuf, acc, ssem):
    c = lax.axis_index("c")
    s = lax.axis_index("s")
    _zero_acc(s, zbuf, acc)

    def ones_row(i, carry):
        ones[i, :] = jnp.full((H,), 1.0, jnp.float32)
        return carry

    lax.fori_loop(0, CHUNK, ones_row, 0)
    chunk_off = c * CH_SC + s * CH_SUB
    pltpu.sync_copy(dst_hbm.at[pl.ds(chunk_off, CH_SUB)], didx)
    plsc.subcore_barrier()

    def group(g, carry):
        kb = g * 16
        sd = [pltpu.async_copy(ones, acc.at[didx.at[kb + b]], ssem, add=True)
              for b in range(16)]
        for d in sd:
            d.wait()
        return carry

    lax.fori_loop(0, CH_SUB // 16, group, 0)
    plsc.subcore_barrier()
    _copy_out(c, s, acc, out_hbm)


def _sig(z):
    return 0.5 * jnp.tanh(0.5 * z) + 0.5


def _tc1_body(degp, x, w1, dinv_o, y1_o):
    deg = degp[0] + degp[1] + 1.0
    dinv = lax.rsqrt(deg)
    dinv_o[...] = dinv
    y1_o[...] = jnp.dot(x[...], w1[...], preferred_element_type=jnp.float32) * dinv


def _tc2_body(s1, y1, dinv, b1, w2, x1_o, y2_o):
    di = dinv[...]
    x1 = jnp.maximum(di * (s1[0] + s1[1] + y1[...]) + b1[...], 0.0)
    x1_o[...] = x1
    y2_o[...] = jnp.dot(x1, w2[...], preferred_element_type=jnp.float32) * di


def _tc3_body(s2, y2, dinv, b2, x1r, wif, whf, bif, wib, whb, bib, watt, batt,
              y3_o):
    di = dinv[...]
    x1 = x1r[...]
    x2 = jnp.maximum(di * (s2[0] + s2[1] + y2[...]) + b2[...], 0.0)
    dot = lambda a, b: jnp.dot(a, b, preferred_element_type=jnp.float32)
    g = dot(x1, wif[...]) + bif[...]
    i, f, gg, o = jnp.split(g, 4, axis=-1)
    c1 = _sig(i) * jnp.tanh(gg)
    h1f = _sig(o) * jnp.tanh(c1)
    g = dot(x2, wif[...]) + dot(h1f, whf[...]) + bif[...]
    i, f, gg, o = jnp.split(g, 4, axis=-1)
    c2 = _sig(f) * c1 + _sig(i) * jnp.tanh(gg)
    h2f = _sig(o) * jnp.tanh(c2)
    g = dot(x2, wib[...]) + bib[...]
    i, f, gg, o = jnp.split(g, 4, axis=-1)
    cb1 = _sig(i) * jnp.tanh(gg)
    h1b = _sig(o) * jnp.tanh(cb1)
    g = dot(x1, wib[...]) + dot(h1b, whb[...]) + bib[...]
    i, f, gg, o = jnp.split(g, 4, axis=-1)
    cb2 = _sig(f) * cb1 + _sig(i) * jnp.tanh(gg)
    h2b = _sig(o) * jnp.tanh(cb2)
    wa = watt[...]
    a0 = jnp.sum(jnp.concatenate([h1f, h2b], -1) * wa, -1, keepdims=True) + batt[...]
    a1 = jnp.sum(jnp.concatenate([h2f, h1b], -1) * wa, -1, keepdims=True) + batt[...]
    m = jnp.maximum(a0, a1)
    e0 = jnp.exp(a0 - m)
    e1 = jnp.exp(a1 - m)
    xjk = (e0 * x1 + e1 * x2) / (e0 + e1)
    y3_o[...] = xjk * di


def _tc4_body(s3, y3, dinv, wlin, blin, out_o):
    xp = dinv[...] * (s3[0] + s3[1] + y3[...])
    logits = jnp.dot(xp, wlin[...], preferred_element_type=jnp.float32) + blin[...]
    m = jnp.max(logits, -1, keepdims=True)
    z = logits - m
    out_o[...] = z - jnp.log(jnp.sum(jnp.exp(z), -1, keepdims=True))


def _spec(shape, bn):
    if shape[0] == N:
        blk = (bn,) + shape[1:]
        return pl.BlockSpec(blk, lambda i: (i,) + (0,) * (len(shape) - 1))
    if len(shape) == 3 and shape[1] == N:
        blk = (shape[0], bn, shape[2])
        return pl.BlockSpec(blk, lambda i: (0, i, 0))
    return pl.BlockSpec(shape, lambda i: (0,) * len(shape))


def _tc_call(body, in_shapes, out_shapes, grid=10):
    bn = N // grid
    return pl.pallas_call(
        body,
        grid=(grid,),
        in_specs=[_spec(s, bn) for s in in_shapes],
        out_specs=[_spec(s, bn) for s in out_shapes],
        out_shape=[jax.ShapeDtypeStruct(s, jnp.float32) for s in out_shapes],
    )


def kernel(x, edge_index, W1, b1, W2, b2, W_ih_f, W_hh_f, b_ih_f, b_hh_f,
           W_ih_b, W_hh_b, b_ih_b, b_hh_b, W_att, b_att, W_lin, b_lin):
    pad = E_PAD - E
    src = jnp.concatenate([edge_index[0], jnp.zeros((pad,), jnp.int32)])
    src = src.reshape(CH_TOT, CHUNK)
    dst = jnp.concatenate([edge_index[1], jnp.full((pad,), N, jnp.int32)])
    dst = dst.reshape(CH_TOT, CHUNK)
    NH = (N, H)
    P = (NC, N, H)

    deg_p = _degsum(dst)
    dinv, y1 = _tc_call(_tc1_body, [P, (N, D), (D, H)], [NH, NH])(
        deg_p, x, W1)
    s1 = _segsum(src, dst, y1)
    x1, y2 = _tc_call(_tc2_body, [P, NH, NH, (1, H), (H, H)], [NH, NH])(
        s1, y1, dinv, b1.reshape(1, H), W2)
    s2 = _segsum(src, dst, y2)
    (y3,) = _tc_call(
        _tc3_body,
        [P, NH, NH, (1, H), NH, (H, 128), (32, 128), (1, 128),
         (H, 128), (32, 128), (1, 128), (1, 64), (1, 1)],
        [NH], grid=5)(
        s2, y2, dinv, b2.reshape(1, H), x1,
        W_ih_f.T, W_hh_f.T, (b_ih_f + b_hh_f).reshape(1, 4 * 32),
        W_ih_b.T, W_hh_b.T, (b_ih_b + b_hh_b).reshape(1, 4 * 32),
        W_att, b_att.reshape(1, 1))
    s3 = _segsum(src, dst, y3)
    (out,) = _tc_call(_tc4_body, [P, NH, NH, (H, H), (1, H)], [NH])(
        s3, y3, dinv, W_lin, b_lin.reshape(1, H))
    return out


# speedup vs baseline: 1.0060x; 1.0060x over previous
import functools

import jax
import jax.numpy as jnp
from jax import lax
from jax.experimental import pallas as pl
from jax.experimental.pallas import tpu as pltpu
from jax.experimental.pallas import tpu_sc as plsc

N = 10000
E = 320000
D = 128
H = 16

NC = 2
NS = 16
CHUNK = 128
E_PAD = 327680
CH_TOT = E_PAD // CHUNK
CH_SC = CH_TOT // NC
CH_SUB = CH_SC // NS
GRP = 8
FAST_C = 0
CH_F = 88
CH_S = 72
NG_F = CH_F // GRP
NG_S = CH_S // GRP
CUT = NS * CH_F
NA = N + 8
R_MAIN = 632
R_LAST = N - (NS - 1) * R_MAIN
RZ_LAST = NA - (NS - 1) * R_MAIN

_mesh = plsc.VectorSubcoreMesh(core_axis_name="c", subcore_axis_name="s")
_sc_params = pltpu.CompilerParams(use_tc_tiling_on_sc=False)


def _zero_acc(s, zbuf, acc):
    def zero_row(i, carry):
        zbuf[i, :] = jnp.zeros((H,), jnp.float32)
        return carry

    lax.fori_loop(0, R_MAIN, zero_row, 0)

    @pl.when(s < NS - 1)
    def _():
        pltpu.sync_copy(zbuf, acc.at[pl.ds(s * R_MAIN, R_MAIN)])

    @pl.when(s == NS - 1)
    def _():
        pltpu.sync_copy(zbuf.at[pl.ds(0, RZ_LAST)],
                        acc.at[pl.ds((NS - 1) * R_MAIN, RZ_LAST)])


def _copy_out(c, s, acc, out_hbm):
    @pl.when(s < NS - 1)
    def _():
        pltpu.sync_copy(acc.at[pl.ds(s * R_MAIN, R_MAIN)],
                        out_hbm.at[c, pl.ds(s * R_MAIN, R_MAIN)])

    @pl.when(s == NS - 1)
    def _():
        pltpu.sync_copy(acc.at[pl.ds((NS - 1) * R_MAIN, R_LAST)],
                        out_hbm.at[c, pl.ds((NS - 1) * R_MAIN, R_LAST)])


@functools.partial(
    pl.kernel,
    out_type=jax.ShapeDtypeStruct((NC, N, H), jnp.float32),
    mesh=_mesh,
    compiler_params=_sc_params,
    scratch_types=[
        pltpu.VMEM((CH_F, CHUNK), jnp.int32),
        pltpu.VMEM((CH_F, CHUNK), jnp.int32),
        pltpu.VMEM((GRP, CHUNK, H), jnp.float32),
        pltpu.VMEM((R_MAIN, H), jnp.float32),
        pltpu.VMEM_SHARED((NA, H), jnp.float32),
        pltpu.VMEM_SHARED((N, H), jnp.float32),
        pltpu.SemaphoreType.DMA,
        pltpu.SemaphoreType.DMA,
    ],
)
def _segsum(src_hbm, dst_hbm, y_hbm, out_hbm, sidx, didx, rows, zbuf, acc,
            ytab, gsem, ssem):
    c = lax.axis_index("c")
    s = lax.axis_index("s")
    _zero_acc(s, zbuf, acc)

    @pl.when(s < NS - 1)
    def _():
        pltpu.sync_copy(y_hbm.at[pl.ds(s * R_MAIN, R_MAIN)],
                        ytab.at[pl.ds(s * R_MAIN, R_MAIN)])

    @pl.when(s == NS - 1)
    def _():
        pltpu.sync_copy(y_hbm.at[pl.ds((NS - 1) * R_MAIN, R_LAST)],
                        ytab.at[pl.ds((NS - 1) * R_MAIN, R_LAST)])

    fast = c == FAST_C

    @pl.when(fast)
    def _():
        pltpu.sync_copy(src_hbm.at[pl.ds(s * CH_F, CH_F)], sidx)
        pltpu.sync_copy(dst_hbm.at[pl.ds(s * CH_F, CH_F)], didx)

    @pl.when(jnp.logical_not(fast))
    def _():
        pltpu.sync_copy(src_hbm.at[pl.ds(CUT + s * CH_S, CH_S)],
                        sidx.at[pl.ds(0, CH_S)])
        pltpu.sync_copy(dst_hbm.at[pl.ds(CUT + s * CH_S, CH_S)],
                        didx.at[pl.ds(0, CH_S)])

    plsc.subcore_barrier()
    ngrp = jnp.where(fast, NG_F, NG_S)

    def group(g, carry):
        @pl.when(g < ngrp)
        def _():
            kb = g * GRP
            gd = [pltpu.async_copy(ytab.at[sidx.at[kb + b]], rows.at[b], gsem)
                  for b in range(GRP)]
            sd = []
            for b in range(GRP):
                gd[b].wait()
                sd.append(pltpu.async_copy(rows.at[b],
                                           acc.at[didx.at[kb + b]],
                                           ssem, add=True))
            for d in sd:
                d.wait()
        return carry

    lax.fori_loop(0, NG_F, group, 0)
    plsc.subcore_barrier()
    _copy_out(c, s, acc, out_hbm)


@functools.partial(
    pl.kernel,
    out_type=jax.ShapeDtypeStruct((NC, N, H), jnp.float32),
    mesh=_mesh,
    compiler_params=_sc_params,
    scratch_types=[
        pltpu.VMEM((CH_F, CHUNK), jnp.int32),
        pltpu.VMEM((CHUNK, H), jnp.float32),
        pltpu.VMEM((R_MAIN, H), jnp.float32),
        pltpu.VMEM_SHARED((NA, H), jnp.float32),
        pltpu.SemaphoreType.DMA,
    ],
)
def _degsum(dst_hbm, out_hbm, didx, ones, zbuf, acc, ssem):
    c = lax.axis_index("c")
    s = lax.axis_index("s")
    _zero_acc(s, zbuf, acc)

    def ones_row(i, carry):
        ones[i, :] = jnp.full((H,), 1.0, jnp.float32)
        return carry

    lax.fori_loop(0, CHUNK, ones_row, 0)
    fast = c == FAST_C

    @pl.when(fast)
    def _():
        pltpu.sync_copy(dst_hbm.at[pl.ds(s * CH_F, CH_F)], didx)

    @pl.when(jnp.logical_not(fast))
    def _():
        pltpu.sync_copy(dst_hbm.at[pl.ds(CUT + s * CH_S, CH_S)],
                        didx.at[pl.ds(0, CH_S)])

    plsc.subcore_barrier()
    ngrp = jnp.where(fast, NG_F, NG_S)

    def group(g, carry):
        @pl.when(g < ngrp)
        def _():
            kb = g * GRP
            sd = [pltpu.async_copy(ones, acc.at[didx.at[kb + b]], ssem,
                                   add=True)
                  for b in range(GRP)]
            for d in sd:
                d.wait()
        return carry

    lax.fori_loop(0, NG_F, group, 0)
    plsc.subcore_barrier()
    _copy_out(c, s, acc, out_hbm)


def _sig(z):
    return 0.5 * jnp.tanh(0.5 * z) + 0.5


def _tc1_body(degp, x, w1, dinv_o, y1_o):
    deg = degp[0] + degp[1] + 1.0
    dinv = lax.rsqrt(deg)
    dinv_o[...] = dinv
    y1_o[...] = jnp.dot(x[...], w1[...], preferred_element_type=jnp.float32) * dinv


def _tc2_body(s1, y1, dinv, b1, w2, x1_o, y2_o):
    di = dinv[...]
    x1 = jnp.maximum(di * (s1[0] + s1[1] + y1[...]) + b1[...], 0.0)
    x1_o[...] = x1
    y2_o[...] = jnp.dot(x1, w2[...], preferred_element_type=jnp.float32) * di


def _tc3_body(s2, y2, dinv, b2, x1r, wif, whf, bif, wib, whb, bib, watt, batt,
              y3_o):
    di = dinv[...]
    x1 = x1r[...]
    x2 = jnp.maximum(di * (s2[0] + s2[1] + y2[...]) + b2[...], 0.0)
    dot = lambda a, b: jnp.dot(a, b, preferred_element_type=jnp.float32)
    g = dot(x1, wif[...]) + bif[...]
    i, f, gg, o = jnp.split(g, 4, axis=-1)
    c1 = _sig(i) * jnp.tanh(gg)
    h1f = _sig(o) * jnp.tanh(c1)
    g = dot(x2, wif[...]) + dot(h1f, whf[...]) + bif[...]
    i, f, gg, o = jnp.split(g, 4, axis=-1)
    c2 = _sig(f) * c1 + _sig(i) * jnp.tanh(gg)
    h2f = _sig(o) * jnp.tanh(c2)
    g = dot(x2, wib[...]) + bib[...]
    i, f, gg, o = jnp.split(g, 4, axis=-1)
    cb1 = _sig(i) * jnp.tanh(gg)
    h1b = _sig(o) * jnp.tanh(cb1)
    g = dot(x1, wib[...]) + dot(h1b, whb[...]) + bib[...]
    i, f, gg, o = jnp.split(g, 4, axis=-1)
    cb2 = _sig(f) * cb1 + _sig(i) * jnp.tanh(gg)
    h2b = _sig(o) * jnp.tanh(cb2)
    wa = watt[...]
    a0 = jnp.sum(jnp.concatenate([h1f, h2b], -1) * wa, -1, keepdims=True) + batt[...]
    a1 = jnp.sum(jnp.concatenate([h2f, h1b], -1) * wa, -1, keepdims=True) + batt[...]
    m = jnp.maximum(a0, a1)
    e0 = jnp.exp(a0 - m)
    e1 = jnp.exp(a1 - m)
    xjk = (e0 * x1 + e1 * x2) / (e0 + e1)
    y3_o[...] = xjk * di


def _tc4_body(s3, y3, dinv, wlin, blin, out_o):
    xp = dinv[...] * (s3[0] + s3[1] + y3[...])
    logits = jnp.dot(xp, wlin[...], preferred_element_type=jnp.float32) + blin[...]
    m = jnp.max(logits, -1, keepdims=True)
    z = logits - m
    out_o[...] = z - jnp.log(jnp.sum(jnp.exp(z), -1, keepdims=True))


def _spec(shape, bn):
    if shape[0] == N:
        blk = (bn,) + shape[1:]
        return pl.BlockSpec(blk, lambda i: (i,) + (0,) * (len(shape) - 1))
    if len(shape) == 3 and shape[1] == N:
        blk = (shape[0], bn, shape[2])
        return pl.BlockSpec(blk, lambda i: (0, i, 0))
    return pl.BlockSpec(shape, lambda i: (0,) * len(shape))


def _tc_call(body, in_shapes, out_shapes, grid=10):
    bn = N // grid
    return pl.pallas_call(
        body,
        grid=(grid,),
        in_specs=[_spec(s, bn) for s in in_shapes],
        out_specs=[_spec(s, bn) for s in out_shapes],
        out_shape=[jax.ShapeDtypeStruct(s, jnp.float32) for s in out_shapes],
    )


def kernel(x, edge_index, W1, b1, W2, b2, W_ih_f, W_hh_f, b_ih_f, b_hh_f,
           W_ih_b, W_hh_b, b_ih_b, b_hh_b, W_att, b_att, W_lin, b_lin):
    pad = E_PAD - E
    src = jnp.concatenate([edge_index[0], jnp.zeros((pad,), jnp.int32)])
    src = src.reshape(CH_TOT, CHUNK)
    dst = jnp.concatenate([edge_index[1], jnp.full((pad,), N, jnp.int32)])
    dst = dst.reshape(CH_TOT, CHUNK)
    NH = (N, H)
    P = (NC, N, H)

    deg_p = _degsum(dst)
    dinv, y1 = _tc_call(_tc1_body, [P, (N, D), (D, H)], [NH, NH])(
        deg_p, x, W1)
    s1 = _segsum(src, dst, y1)
    x1, y2 = _tc_call(_tc2_body, [P, NH, NH, (1, H), (H, H)], [NH, NH])(
        s1, y1, dinv, b1.reshape(1, H), W2)
    s2 = _segsum(src, dst, y2)
    (y3,) = _tc_call(
        _tc3_body,
        [P, NH, NH, (1, H), NH, (H, 128), (32, 128), (1, 128),
         (H, 128), (32, 128), (1, 128), (1, 64), (1, 1)],
        [NH], grid=5)(
        s2, y2, dinv, b2.reshape(1, H), x1,
        W_ih_f.T, W_hh_f.T, (b_ih_f + b_hh_f).reshape(1, 4 * 32),
        W_ih_b.T, W_hh_b.T, (b_ih_b + b_hh_b).reshape(1, 4 * 32),
        W_att, b_att.reshape(1, 1))
    s3 = _segsum(src, dst, y3)
    (out,) = _tc_call(_tc4_body, [P, NH, NH, (H, H), (1, H)], [NH])(
        s3, y3, dinv, W_lin, b_lin.reshape(1, H))
    return out
